# baseline (speedup 1.0000x reference)
import functools
import math

import jax
import jax.numpy as jnp
from jax import lax
from jax.experimental import pallas as pl
from jax.experimental.pallas import tpu as pltpu
from jax.experimental.pallas import tpu_sc as plsc

_SCALE = 1.0 / math.sqrt(12.0)
_L = 12
_N = 1024
_BLK = 256

_NW = 32
_WROWS = _N // _NW
_CH = 4


def _tc_body(x_ref, q_ref):
    for k in range(_L):
        m = x_ref[k] >= 0.0
        q_ref[k] = jnp.where(m, jnp.float32(_SCALE), jnp.float32(-_SCALE))


def _tc_quant(xt):
    return pl.pallas_call(
        _tc_body,
        grid=(_N // _BLK,),
        in_specs=[pl.BlockSpec((_L, _BLK, _N), lambda i: (0, i, 0))],
        out_specs=pl.BlockSpec((_L, _BLK, _N), lambda i: (0, i, 0)),
        out_shape=jax.ShapeDtypeStruct((_L, _N, _N), jnp.float32),
    )(xt)


def _sc_tokens_body(x_hbm, t_hbm, xbuf, tbuf, sem):
    wid = lax.axis_index("s") * 2 + lax.axis_index("c")
    row0 = wid * _WROWS

    def chunk(c, _):
        r0 = row0 + c * _CH
        pltpu.sync_copy(x_hbm.at[:, pl.ds(r0, _CH), :], xbuf)

        def group(gi, _):
            i = gi // (_N // 16)
            g = gi % (_N // 16)
            acc = jnp.zeros((16,), jnp.int32)
            for k in range(_L):
                x = xbuf[k, i, pl.ds(g * 16, 16)]
                acc = acc + jnp.where(x >= 0.0, jnp.int32(1 << k), jnp.int32(0))
            tbuf[i, pl.ds(g * 16, 16)] = acc
            return 0

        lax.fori_loop(0, _CH * (_N // 16), group, 0)
        pltpu.sync_copy(tbuf, t_hbm.at[pl.ds(r0, _CH), :])
        return 0

    lax.fori_loop(0, _WROWS // _CH, chunk, 0)


_sc_tokens = functools.partial(
    pl.kernel,
    out_type=jax.ShapeDtypeStruct((_N, _N), jnp.int32),
    mesh=plsc.VectorSubcoreMesh(core_axis_name="c", subcore_axis_name="s"),
    scratch_types=[
        pltpu.VMEM((_L, _CH, _N), jnp.float32),
        pltpu.VMEM((_CH, _N), jnp.int32),
        pltpu.SemaphoreType.DMA,
    ],
)(_sc_tokens_body)


def kernel(latents):
    xt = jnp.transpose(latents, (2, 0, 1))
    q3 = _tc_quant(xt)
    t = _sc_tokens(xt)
    quant = jnp.transpose(q3, (1, 2, 0))
    tokens = t.astype(jnp.int64)
    return (quant, tokens)
